# baseline (speedup 1.0000x reference)
import functools

import jax
import jax.numpy as jnp
from jax import lax
from jax.experimental import pallas as pl
from jax.experimental.pallas import tpu as pltpu
from jax.experimental.pallas import tpu_sc as plsc

_N = 20000
_C = 21
_PAD = 300
_THR = 0.5
_KMAX = 512
_MD = 14 * 14 * 21


def _stage_a(meta_ref, prop_ref, dx_ref, dy_ref, dw_ref, dh_ref, sc_ref,
             boxes_ref, msc_ref):
    img_h = meta_ref[0, 0]
    img_w = meta_ref[0, 1]
    scale = meta_ref[0, 2]
    prop = prop_ref[...] / scale
    x1 = prop[:, 0]
    y1 = prop[:, 1]
    w = prop[:, 2] - x1 + 1.0
    h = prop[:, 3] - y1 + 1.0
    cx = x1 + 0.5 * w
    cy = y1 + 0.5 * h

    sc = sc_ref[...]
    nb = sc.shape[0]
    top = jnp.argmax(sc, axis=1)
    oh = lax.broadcasted_iota(jnp.int32, (nb, _C), 1) == top[:, None]
    zf = jnp.zeros((nb, _C), jnp.float32)
    dx = jnp.sum(jnp.where(oh, dx_ref[...], zf), axis=1)
    dy = jnp.sum(jnp.where(oh, dy_ref[...], zf), axis=1)
    dw = jnp.sum(jnp.where(oh, dw_ref[...], zf), axis=1)
    dh = jnp.sum(jnp.where(oh, dh_ref[...], zf), axis=1)

    pcx = dx * w + cx
    pcy = dy * h + cy
    pw = jnp.exp(dw) * w
    ph = jnp.exp(dh) * h
    px1 = jnp.clip(pcx - 0.5 * pw, 0.0, img_w - 1.0)
    py1 = jnp.clip(pcy - 0.5 * ph, 0.0, img_h - 1.0)
    px2 = jnp.clip(pcx + 0.5 * pw, 0.0, img_w - 1.0)
    py2 = jnp.clip(pcy + 0.5 * ph, 0.0, img_h - 1.0)

    boxes_ref[...] = jnp.stack([px1, py1, px2, py2], axis=1)
    msc_ref[...] = jnp.max(sc[:, 1:], axis=1)[:, None]


def _nms(bs_ref, sel_ref, cnt_ref, kb_ref):
    lanes = lax.broadcasted_iota(jnp.int32, (1, _KMAX), 1)
    big = jnp.float32(3e8)
    kx1 = jnp.full((1, _KMAX), big, jnp.float32)
    ky1 = jnp.full((1, _KMAX), big, jnp.float32)
    kx2 = jnp.full((1, _KMAX), -big, jnp.float32)
    ky2 = jnp.full((1, _KMAX), -big, jnp.float32)
    kar = (kx2 - kx1 + 1.0) * (ky2 - ky1 + 1.0)
    sel = jnp.zeros((1, _KMAX), jnp.int32)

    def cond(st):
        i, cnt = st[0], st[1]
        return (i < _N) & (cnt < _PAD)

    def body(st):
        i, cnt, kx1, ky1, kx2, ky2, kar, sel = st
        row = bs_ref[pl.ds(i, 1), :]
        bx1 = jnp.broadcast_to(row[:, 0:1], (1, _KMAX))
        by1 = jnp.broadcast_to(row[:, 1:2], (1, _KMAX))
        bx2 = jnp.broadcast_to(row[:, 2:3], (1, _KMAX))
        by2 = jnp.broadcast_to(row[:, 3:4], (1, _KMAX))
        ar_i = (bx2 - bx1 + 1.0) * (by2 - by1 + 1.0)
        xx1 = jnp.maximum(bx1, kx1)
        yy1 = jnp.maximum(by1, ky1)
        xx2 = jnp.minimum(bx2, kx2)
        yy2 = jnp.minimum(by2, ky2)
        iw = jnp.maximum(xx2 - xx1 + 1.0, 0.0)
        ih = jnp.maximum(yy2 - yy1 + 1.0, 0.0)
        inter = iw * ih
        iou = inter / (ar_i + kar - inter)
        keepit = ~jnp.any(iou > _THR)
        ins = (lanes == cnt) & keepit
        kx1 = jnp.where(ins, bx1, kx1)
        ky1 = jnp.where(ins, by1, ky1)
        kx2 = jnp.where(ins, bx2, kx2)
        ky2 = jnp.where(ins, by2, ky2)
        kar = jnp.where(ins, ar_i, kar)
        sel = jnp.where(ins, i, sel)
        return (i + 1, cnt + keepit.astype(jnp.int32), kx1, ky1, kx2, ky2,
                kar, sel)

    st = lax.while_loop(
        cond, body,
        (jnp.int32(0), jnp.int32(0), kx1, ky1, kx2, ky2, kar, sel))
    sel_ref[...] = st[7]
    cnt_ref[0, 0] = st[1]
    vf = (lanes < st[1]).astype(jnp.float32)
    kb_ref[...] = jnp.concatenate(
        [st[2] * vf, st[3] * vf, st[4] * vf, st[5] * vf], axis=0)


_GW = 19
_BPW = 16
_GB = _GW * _BPW


def _sc_gather(idx_hbm, cnt_hbm, m_hbm, sc_hbm, om_hbm, os_hbm,
               idx_v, mrows_v, srows_v, cnt_v, sem1, sem2):
    wid = lax.axis_index("s") * 2 + lax.axis_index("c")
    base = wid * _BPW

    @pl.when(wid < _GW)
    def _():
        pltpu.sync_copy(idx_hbm.at[pl.ds(base, _BPW)], idx_v)
        idxvec = idx_v[...]
        handles = []
        for r in range(_BPW):
            row = idxvec[r]
            handles.append(pltpu.async_copy(
                m_hbm.at[pl.ds(row, 1)], mrows_v.at[pl.ds(r, 1)], sem1))
            handles.append(pltpu.async_copy(
                sc_hbm.at[pl.ds(row, 1)], srows_v.at[pl.ds(r, 1)], sem2))
        pltpu.sync_copy(cnt_hbm, cnt_v)
        for h in handles:
            h.wait()
        cntv = cnt_v[...]
        for r in range(_BPW):
            vf = jnp.where(base + r < cntv, 1.0, 0.0).astype(jnp.float32)

            def mb(c, carry, r=r, vf=vf):
                for u in range(4):
                    sl = pl.ds((c * 4 + u) * 16, 16)
                    mrows_v[r, sl] = mrows_v[r, sl] * vf
                return carry

            lax.fori_loop(0, 64, mb, 0)
            sl = pl.ds(4096, 16)
            mrows_v[r, sl] = mrows_v[r, sl] * vf
            sl = pl.ds(_MD - 16, 16)
            mrows_v[r, sl] = mrows_v[r, sl] * vf
            sl = pl.ds(0, 16)
            srows_v[r, sl] = srows_v[r, sl] * vf
            sl = pl.ds(_C - 16, 16)
            srows_v[r, sl] = srows_v[r, sl] * vf
        pltpu.sync_copy(mrows_v, om_hbm.at[pl.ds(base, _BPW)])
        pltpu.sync_copy(srows_v, os_hbm.at[pl.ds(base, _BPW)])


def kernel(metadata, deltas, proposals, scores, masks):
    prop = proposals[0]
    d4 = deltas[0].reshape(_N, _C, 4)
    sc = scores[0]

    bn = 2000
    boxes_top, msc = pl.pallas_call(
        _stage_a,
        grid=(_N // bn,),
        in_specs=[
            pl.BlockSpec(memory_space=pltpu.SMEM),
            pl.BlockSpec((bn, 4), lambda i: (i, 0)),
            pl.BlockSpec((bn, _C), lambda i: (i, 0)),
            pl.BlockSpec((bn, _C), lambda i: (i, 0)),
            pl.BlockSpec((bn, _C), lambda i: (i, 0)),
            pl.BlockSpec((bn, _C), lambda i: (i, 0)),
            pl.BlockSpec((bn, _C), lambda i: (i, 0)),
        ],
        out_specs=[
            pl.BlockSpec((bn, 4), lambda i: (i, 0)),
            pl.BlockSpec((bn, 1), lambda i: (i, 0)),
        ],
        out_shape=[
            jax.ShapeDtypeStruct((_N, 4), jnp.float32),
            jax.ShapeDtypeStruct((_N, 1), jnp.float32),
        ],
    )(metadata, prop, d4[..., 0], d4[..., 1], d4[..., 2], d4[..., 3], sc)

    order = jnp.argsort(-msc[:, 0])
    bs = boxes_top[order]

    sel, cnt, kb = pl.pallas_call(
        _nms,
        in_specs=[pl.BlockSpec(memory_space=pltpu.VMEM)],
        out_specs=[
            pl.BlockSpec(memory_space=pltpu.VMEM),
            pl.BlockSpec(memory_space=pltpu.SMEM),
            pl.BlockSpec(memory_space=pltpu.VMEM),
        ],
        out_shape=[
            jax.ShapeDtypeStruct((1, _KMAX), jnp.int32),
            jax.ShapeDtypeStruct((1, 1), jnp.int32),
            jax.ShapeDtypeStruct((4, _KMAX), jnp.float32),
        ],
    )(bs)

    idx = order[sel[0, :_PAD]]
    idxp = jnp.concatenate([idx, jnp.zeros((_GB - _PAD,), idx.dtype)])
    cnt16 = jnp.broadcast_to(cnt.reshape(1), (16,))

    mesh = plsc.VectorSubcoreMesh(core_axis_name="c", subcore_axis_name="s")
    om_pad, os_pad = pl.kernel(
        _sc_gather,
        mesh=mesh,
        out_type=[
            jax.ShapeDtypeStruct((_GB, _MD), jnp.float32),
            jax.ShapeDtypeStruct((_GB, _C), jnp.float32),
        ],
        scratch_types=[
            pltpu.VMEM((_BPW,), jnp.int32),
            pltpu.VMEM((_BPW, _MD), jnp.float32),
            pltpu.VMEM((_BPW, _C), jnp.float32),
            pltpu.VMEM((16,), jnp.int32),
            pltpu.SemaphoreType.DMA,
            pltpu.SemaphoreType.DMA,
        ],
    )(idxp, cnt16, masks[0].reshape(_N, _MD), sc)

    ob = jnp.transpose(kb)[:_PAD]
    return (ob[None], os_pad[:_PAD][None],
            om_pad[:_PAD].reshape(1, _PAD, 14, 14, 21))
